# baseline (speedup 1.0000x reference)
import functools

import jax
import jax.numpy as jnp
from jax import lax
from jax.experimental import pallas as pl
from jax.experimental.pallas import tpu as pltpu
from jax.experimental.pallas import tpu_sc as plsc

NUM_WEIGHTS = 512
PADDING_IDX = 1
LANES = 16
NUM_WORKERS = 32


def _build(T, D, V):
    R = V // NUM_WORKERS
    NG = T // LANES
    mesh = plsc.VectorSubcoreMesh(core_axis_name="c", subcore_axis_name="s")

    @functools.partial(
        pl.kernel,
        mesh=mesh,
        compiler_params=pltpu.CompilerParams(needs_layout_passes=False),
        out_type=jax.ShapeDtypeStruct((V, D), jnp.float32),
        scratch_types=[
            pltpu.VMEM((T,), jnp.int32),
            pltpu.VMEM((T,), jnp.int32),
            pltpu.VMEM((T,), jnp.int32),
            pltpu.VMEM((T, D), jnp.float32),
            pltpu.VMEM((R, D), jnp.float32),
            pltpu.VMEM((LANES,), jnp.float32),
            pltpu.SemaphoreType.DMA,
        ],
    )
    def k(grad_hbm, idx_hbm, out_hbm,
          idx_v, tlist_v, rlist_v, slots_v, acc_v, cnt_v, sem):
        wid = lax.axis_index("s") * 2 + lax.axis_index("c")
        base = wid * R

        pltpu.sync_copy(idx_hbm.at[0], idx_v)

        def tz_body(g, _):
            tlist_v[pl.ds(g * LANES, LANES)] = jnp.zeros((LANES,), jnp.int32)
            return 0
        lax.fori_loop(0, NG, tz_body, 0, unroll=NG)
        cnt_v[...] = jnp.zeros((LANES,), jnp.float32)

        lanes_iota = lax.broadcasted_iota(jnp.int32, (LANES,), 0)
        row_ids = base + lanes_iota

        def compact_body(g, n):
            rvec = idx_v[pl.ds(g * LANES, LANES)]
            tvec = g * LANES + lanes_iota
            owned = ((rvec >= base) & (rvec < base + R)
                     & (rvec != PADDING_IDX))
            plsc.store_compressed(tlist_v.at[pl.ds(n, LANES)], tvec, mask=owned)
            plsc.store_compressed(rlist_v.at[pl.ds(n, LANES)], rvec, mask=owned)
            return n + plsc.all_reduce_population_count(owned)[0]
        n = lax.fori_loop(0, NG, compact_body, 0)
        nch = (n + LANES - 1) // LANES

        def fire_body(ch, _):
            @pl.when(ch < nch)
            def _(ch=ch):
                pltpu.async_copy(
                    grad_hbm.at[tlist_v.at[pl.ds(ch * LANES, LANES)]],
                    slots_v.at[pl.ds(ch * LANES, LANES)], sem)
            return 0
        lax.fori_loop(0, NG, fire_body, 0)

        for i in range(R):
            def zero_body(j, _, i=i):
                acc_v[i, pl.ds(j * LANES, LANES)] = jnp.zeros(
                    (LANES,), jnp.float32)
                return 0
            lax.fori_loop(0, D // LANES, zero_body, 0, unroll=8)

        def drain_body(ch, _):
            @pl.when(ch < nch)
            def _(ch=ch):
                pltpu.make_async_copy(
                    grad_hbm.at[tlist_v.at[pl.ds(ch * LANES, LANES)]],
                    slots_v.at[pl.ds(ch * LANES, LANES)], sem).wait()
                rvecc = rlist_v[pl.ds(ch * LANES, LANES)]
                for lane in range(LANES):
                    kk = ch * LANES + lane

                    @pl.when(kk < n)
                    def _(kk=kk, r=rvecc[lane]):
                        cnt_v[...] = cnt_v[...] + jnp.where(
                            row_ids == r, 1.0, 0.0)
                        loc = r - base

                        def acc_body(j, _, loc=loc, kk=kk):
                            sl = pl.ds(j * LANES, LANES)
                            acc_v[loc, sl] = (acc_v[loc, sl]
                                              + slots_v[kk, sl])
                            return 0
                        lax.fori_loop(0, D // LANES, acc_body, 0, unroll=4)
            return 0
        lax.fori_loop(0, NG, drain_body, 0)
        cnt16 = cnt_v[...]

        inv16 = 1.0 / jnp.maximum(cnt16, 1.0)
        for i in range(R):
            @pl.when(cnt16[i] > 1.0)
            def _(i=i):
                iv = inv16[i]

                def sc_body(j, _, i=i, iv=iv):
                    sl = pl.ds(j * LANES, LANES)
                    acc_v[i, sl] = acc_v[i, sl] * iv
                    return 0
                lax.fori_loop(0, D // LANES, sc_body, 0, unroll=3)

        pltpu.sync_copy(acc_v, out_hbm.at[pl.ds(base, R)])

    return k


def kernel(grad_output, index):
    T = index.shape[0] * index.shape[1]
    D = grad_output.shape[-1]
    idx = index.astype(jnp.int32)
    return _build(T, D, NUM_WEIGHTS)(grad_output[0], idx)
